# baseline (speedup 1.0000x reference)
import functools

import jax
import jax.numpy as jnp
from jax import lax
from jax.experimental import pallas as pl
from jax.experimental.pallas import tpu as pltpu
from jax.experimental.pallas import tpu_sc as plsc

N = 10000
E = 320000
D = 128

NC = 2
NS = 16
LANES = 16

NPAD = 10240
DUMP_ROW = 10000
CHUNK = 48
E_PER_TILE = 21504
E_PAD = NS * E_PER_TILE
NCH = E_PER_TILE // CHUNK
NIBP = E_PER_TILE // 256
K = 79
ACC_ROWS = 10112
ACC_PER_TILE = ACC_ROWS // NS

P_CHUNK = 48
PPAD = 12288
P_PER_W = PPAD // (NC * NS)
ROWS_PER_TILE = NPAD // NS


def _sc_edge_kernel(table, eidx_hbm, par_idx, zinit,
                    outc, outo, outp,
                    eidx, recs, counts, cursors, sbuf, dbuf, rows, acc,
                    zsem, isem0, isem1,
                    gsem0, gsem1, gsem2, gsem3,
                    ssem0, ssem1, ssem2, ssem3, sem):
    c = lax.axis_index("c")
    s = lax.axis_index("s")
    isems = (isem0, isem1)
    gsems = (gsem0, gsem1, gsem2, gsem3)
    ssems = (ssem0, ssem1, ssem2, ssem3)
    i32 = jnp.int32

    zr = s * ACC_PER_TILE
    pltpu.async_copy(zinit.at[pl.ds(zr, ACC_PER_TILE)],
                     acc.at[pl.ds(zr, ACC_PER_TILE)], zsem)

    tid = c * NS + s
    lane = lax.iota(i32, LANES)
    ones = jnp.ones((LANES,), jnp.float32)

    def zb(i, carry):
        counts[pl.ds(i * LANES, LANES)] = jnp.zeros((LANES,), jnp.float32)
        return carry

    lax.fori_loop(0, K, zb, 0)

    def issue_blk(blk, e):
        pltpu.async_copy(eidx_hbm.at[tid, blk], eidx.at[e], isems[e])

    def wait_blk(e):
        pltpu.make_async_copy(eidx_hbm.at[tid, 0], eidx.at[e],
                              isems[e]).wait()

    issue_blk(0, 0)
    issue_blk(1, 1)

    def hist_body(ib2, carry):
        for e in range(2):
            blk = ib2 * 2 + e
            wait_blk(e)
            for j in range(2):
                for h in range(8):
                    r = eidx[e, j, pl.ds(h * LANES, LANES)]
                    fi = lax.shift_right_logical(r, 22) * LANES + lane
                    v = plsc.load_gather(counts, [fi])
                    plsc.store_scatter(counts, [fi], v + 1.0)

            @pl.when(blk + 2 < NIBP)
            def _():
                issue_blk(blk + 2, e)
        return carry

    lax.fori_loop(0, NIBP // 2, hist_body, 0)

    def pfx(i, run):
        v = counts[pl.ds(i * LANES, LANES)]
        inc = plsc.cumsum(v)
        cursors[pl.ds(i * LANES, LANES)] = (inc - v) + run
        return run + jnp.sum(v)

    lax.fori_loop(0, K, pfx, jnp.zeros((), jnp.float32))

    issue_blk(0, 0)
    issue_blk(1, 1)

    def place_body(ib2, carry):
        for e in range(2):
            blk = ib2 * 2 + e
            wait_blk(e)
            for j in range(2):
                for h in range(8):
                    r = eidx[e, j, pl.ds(h * LANES, LANES)]
                    fi = lax.shift_right_logical(r, 22) * LANES + lane
                    pos_f = plsc.load_gather(cursors, [fi])
                    plsc.store_scatter(cursors, [fi], pos_f + 1.0)
                    pos = pos_f.astype(i32)
                    plsc.store_scatter(
                        recs,
                        [lax.shift_right_logical(pos, 7),
                         lax.bitwise_and(pos, 127)], r)

            @pl.when(blk + 2 < NIBP)
            def _():
                issue_blk(blk + 2, e)
        return carry

    lax.fori_loop(0, NIBP // 2, place_body, 0)

    pltpu.make_async_copy(zinit.at[pl.ds(zr, ACC_PER_TILE)],
                          acc.at[pl.ds(zr, ACC_PER_TILE)], zsem).wait()
    plsc.subcore_barrier()

    def unpack(k, m):
        for h in range(CHUNK // LANES):
            f = k * CHUNK + h * LANES
            r = recs[f >> 7, pl.ds(f & 127, LANES)]
            sbuf[m, pl.ds(h * LANES, LANES)] = lax.shift_right_logical(r, 14)
            dbuf[m, pl.ds(h * LANES, LANES)] = lax.bitwise_and(r, 16383)

    def issue_gather(m, b):
        pltpu.async_copy(table.at[sbuf.at[m]], rows.at[b], gsems[b])

    def wait_gather(b):
        pltpu.make_async_copy(table.at[sbuf.at[0]], rows.at[b],
                              gsems[b]).wait()

    def issue_scatter(m, b):
        pltpu.async_copy(rows.at[b], acc.at[dbuf.at[m]], ssems[b], add=True)

    def wait_scatter(b):
        pltpu.make_async_copy(rows.at[b], acc.at[dbuf.at[0]],
                              ssems[b]).wait()

    unpack(0, 0)
    unpack(1, 1)
    issue_gather(0, 0)
    issue_gather(1, 1)

    def body(it, carry):
        for u in range(4):
            cc = it * 4 + u
            b = u % 4
            bp = (u + 2) % 4

            @pl.when(cc >= 2)
            def _():
                wait_scatter(bp)

            @pl.when(cc + 2 < NCH)
            def _():
                unpack(cc + 2, bp)
                issue_gather(bp, bp)

            wait_gather(b)
            issue_scatter(b, b)
        return carry

    lax.fori_loop(0, NCH // 4, body, 0)
    wait_scatter((NCH - 2) % 4)
    wait_scatter((NCH - 1) % 4)
    plsc.subcore_barrier()

    @pl.when(c == 0)
    def _():
        pltpu.sync_copy(acc.at[pl.ds(zr, ACC_PER_TILE)],
                        outc.at[pl.ds(zr, ACC_PER_TILE)])

    @pl.when(c == 1)
    def _():
        pltpu.sync_copy(acc.at[pl.ds(zr, ACC_PER_TILE)],
                        outo.at[pl.ds(zr, ACC_PER_TILE)])

    wid = s * NC + c
    pbase = wid * P_PER_W

    def pbody(k, carry):
        off = pbase + k * P_CHUNK
        pltpu.sync_copy(par_idx.at[pl.ds(off, P_CHUNK)], sbuf.at[0])
        pltpu.async_copy(table.at[sbuf.at[0]], rows.at[0], sem).wait()
        pltpu.sync_copy(rows.at[0], outp.at[pl.ds(off, P_CHUNK)])
        return carry

    lax.fori_loop(0, P_PER_W // P_CHUNK, pbody, 0)


_sc_call = functools.partial(
    pl.kernel,
    out_type=(
        jax.ShapeDtypeStruct((NPAD, D), jnp.float32),
        jax.ShapeDtypeStruct((NPAD, D), jnp.float32),
        jax.ShapeDtypeStruct((PPAD, D), jnp.float32),
    ),
    mesh=plsc.VectorSubcoreMesh(core_axis_name="c", subcore_axis_name="s",
                                num_cores=NC, num_subcores=NS),
    compiler_params=pltpu.CompilerParams(needs_layout_passes=False),
    scratch_types=[
        pltpu.VMEM((2, 2, 8 * LANES), jnp.int32),
        pltpu.VMEM((E_PER_TILE // 128, 128), jnp.int32),
        pltpu.VMEM((K * LANES,), jnp.float32),
        pltpu.VMEM((K * LANES,), jnp.float32),
        pltpu.VMEM((4, CHUNK), jnp.int32),
        pltpu.VMEM((4, CHUNK), jnp.int32),
        pltpu.VMEM((4, CHUNK, D), jnp.float32),
        pltpu.VMEM_SHARED((ACC_ROWS, D), jnp.float32),
    ] + [pltpu.SemaphoreType.DMA] * 12,
)(_sc_edge_kernel)


def _elu(x):
    return jnp.where(x > 0, x, jnp.exp(jnp.minimum(x, 0.0)) - 1.0)


BN = 512
GRID = NPAD // BN


def _tc_mlp_kernel(xpar, xch, xop, xself, w1, w2, w3, wc1, wc2, wc3, ball,
                   out_ref):
    i = pl.program_id(0)
    f32 = jnp.float32

    def mlp(x, j):
        h = _elu(jnp.dot(x, w1[j], preferred_element_type=f32) + ball[j, 0])
        h = _elu(jnp.dot(h, w2[j], preferred_element_type=f32) + ball[j, 1])
        return jnp.dot(h, w3[j], preferred_element_type=f32) + ball[j, 2]

    e_par = mlp(xpar[...], 0)
    e_ch = mlp(xch[...], 1)
    e_op = mlp(xop[...], 2)
    e_self = mlp(xself[...], 3)

    wc1v = wc1[...]
    h = (jnp.dot(e_par, wc1v[0:128], preferred_element_type=f32)
         + jnp.dot(e_ch, wc1v[128:256], preferred_element_type=f32)
         + jnp.dot(e_op, wc1v[256:384], preferred_element_type=f32)
         + jnp.dot(e_self, wc1v[384:512], preferred_element_type=f32)
         + ball[4, 0])
    h = _elu(h)
    h = _elu(jnp.dot(h, wc2[...], preferred_element_type=f32) + ball[4, 1])
    y = jnp.dot(h, wc3[...], preferred_element_type=f32) + ball[4, 2]

    row = i * BN + lax.broadcasted_iota(jnp.int32, (BN, D), 0)
    out_ref[...] = jnp.where(row == (N - 1), 0.0, y)


def _tc_call(xpar, xch, xop, xself, w1, w2, w3, wc1, wc2, wc3, ball):
    full3 = pl.BlockSpec((4, D, D), lambda i: (0, 0, 0))
    blk = pl.BlockSpec((BN, D), lambda i: (i, 0))
    return pl.pallas_call(
        _tc_mlp_kernel,
        grid=(GRID,),
        in_specs=[blk, blk, blk, blk,
                  full3, full3, full3,
                  pl.BlockSpec((4 * D, D), lambda i: (0, 0)),
                  pl.BlockSpec((D, D), lambda i: (0, 0)),
                  pl.BlockSpec((D, D), lambda i: (0, 0)),
                  pl.BlockSpec((5, 3, D), lambda i: (0, 0, 0))],
        out_specs=blk,
        out_shape=jax.ShapeDtypeStruct((NPAD, D), jnp.float32),
    )(xpar, xch, xop, xself, w1, w2, w3, wc1, wc2, wc3, ball)


def kernel(items, operations, parents, item_assembly_edge_index,
           operation_assembly_edge_index, self_p, parent_p, children_p,
           ops_p, comb_p):
    i32 = jnp.int32
    table = jnp.concatenate([items, operations], axis=0)

    ii = item_assembly_edge_index.astype(i32)
    oi = operation_assembly_edge_index.astype(i32)
    pad_e = E_PAD - E
    src_all = jnp.concatenate([
        ii[1], jnp.zeros((pad_e,), i32),
        oi[1] + N, jnp.zeros((pad_e,), i32),
    ])
    dst_all = jnp.concatenate([
        ii[0], jnp.full((pad_e,), DUMP_ROW, i32),
        oi[0], jnp.full((pad_e,), DUMP_ROW, i32),
    ])
    eidx_hbm = (src_all * 16384 + dst_all).reshape(NC * NS, NIBP, 2, 128)
    par_idx = jnp.concatenate([parents.astype(i32),
                               jnp.zeros((PPAD - N,), i32)])
    zinit = jnp.zeros((NPAD, D), jnp.float32)

    aggc, aggo, par_rows = _sc_call(table, eidx_hbm, par_idx, zinit)
    par_rows = par_rows[:NPAD]

    w1 = jnp.stack([parent_p[0], children_p[0], ops_p[0], self_p[0]])
    w2 = jnp.stack([parent_p[2], children_p[2], ops_p[2], self_p[2]])
    w3 = jnp.stack([parent_p[4], children_p[4], ops_p[4], self_p[4]])
    ball = jnp.stack([
        jnp.stack([parent_p[1], parent_p[3], parent_p[5]]),
        jnp.stack([children_p[1], children_p[3], children_p[5]]),
        jnp.stack([ops_p[1], ops_p[3], ops_p[5]]),
        jnp.stack([self_p[1], self_p[3], self_p[5]]),
        jnp.stack([comb_p[1], comb_p[3], comb_p[5]]),
    ])

    items_pad = jnp.pad(items, ((0, NPAD - N), (0, 0)))
    y = _tc_call(par_rows, aggc, aggo, items_pad,
                 w1, w2, w3, comb_p[0], comb_p[2], comb_p[4], ball)
    return y[:N]


# speedup vs baseline: 2.2758x; 2.2758x over previous
import functools

import jax
import jax.numpy as jnp
from jax import lax
from jax.experimental import pallas as pl
from jax.experimental.pallas import tpu as pltpu
from jax.experimental.pallas import tpu_sc as plsc

N = 10000
E = 320000
D = 128

NC = 2
NS = 16

NPAD = 10240
DUMP_ROW = 10000
CHUNK = 128
E_PER_TILE = 20480
E_PAD = NS * E_PER_TILE
CH_PER_TILE = E_PER_TILE // CHUNK
P_CHUNK = 128
PPAD = 12288
P_PER_W = PPAD // (NC * NS)
ROWS_PER_TILE = NPAD // NS

IB = 16
NIB = CH_PER_TILE // IB


def _sc_edge_kernel(table, eidx_hbm, par_idx, zinit,
                    outc, outo, outp,
                    eidx, rows, pidx, acc,
                    isem0, isem1, gsem0, gsem1, sem):
    c = lax.axis_index("c")
    s = lax.axis_index("s")
    isems = (isem0, isem1)
    gsems = (gsem0, gsem1)

    zr = s * ROWS_PER_TILE
    pltpu.sync_copy(zinit.at[pl.ds(zr, ROWS_PER_TILE)],
                    acc.at[pl.ds(zr, ROWS_PER_TILE)])
    plsc.subcore_barrier()

    tid = c * NS + s

    def issue_idx(blk, e):
        pltpu.async_copy(eidx_hbm.at[tid, blk], eidx.at[e], isems[e])

    def issue_gather(e, j, b):
        pltpu.async_copy(table.at[eidx.at[e, j, 0]], rows.at[b], gsems[b])

    def wait_gather(b):
        pltpu.make_async_copy(table.at[eidx.at[0, 0, 0]], rows.at[b],
                              gsems[b]).wait()

    issue_idx(0, 0)
    issue_idx(1, 1)

    def body(ib2, carry):
        for e in range(2):
            blk = ib2 * 2 + e
            pltpu.make_async_copy(eidx_hbm.at[tid, 0], eidx.at[e],
                                  isems[e]).wait()
            issue_gather(e, 0, 0)
            issue_gather(e, 1, 1)
            for j in range(IB):
                b = j % 2
                wait_gather(b)
                pltpu.sync_copy(rows.at[b], acc.at[eidx.at[e, j, 1]],
                                add=True)
                if j + 2 < IB:
                    issue_gather(e, j + 2, b)

            @pl.when(blk + 2 < NIB)
            def _():
                issue_idx(blk + 2, e)
        return carry

    lax.fori_loop(0, NIB // 2, body, 0)
    plsc.subcore_barrier()

    @pl.when(c == 0)
    def _():
        pltpu.sync_copy(acc.at[pl.ds(zr, ROWS_PER_TILE)],
                        outc.at[pl.ds(zr, ROWS_PER_TILE)])

    @pl.when(c == 1)
    def _():
        pltpu.sync_copy(acc.at[pl.ds(zr, ROWS_PER_TILE)],
                        outo.at[pl.ds(zr, ROWS_PER_TILE)])

    wid = s * NC + c
    pbase = wid * P_PER_W

    def pbody(k, carry):
        off = pbase + k * P_CHUNK
        pltpu.sync_copy(par_idx.at[pl.ds(off, P_CHUNK)], pidx)
        pltpu.async_copy(table.at[pidx], rows.at[0], sem).wait()
        pltpu.sync_copy(rows.at[0], outp.at[pl.ds(off, P_CHUNK)])
        return carry

    lax.fori_loop(0, P_PER_W // P_CHUNK, pbody, 0)


_sc_call = functools.partial(
    pl.kernel,
    out_type=(
        jax.ShapeDtypeStruct((NPAD, D), jnp.float32),
        jax.ShapeDtypeStruct((NPAD, D), jnp.float32),
        jax.ShapeDtypeStruct((PPAD, D), jnp.float32),
    ),
    mesh=plsc.VectorSubcoreMesh(core_axis_name="c", subcore_axis_name="s",
                                num_cores=NC, num_subcores=NS),
    scratch_types=[
        pltpu.VMEM((2, IB, 2, CHUNK), jnp.int32),
        pltpu.VMEM((2, CHUNK, D), jnp.float32),
        pltpu.VMEM((P_CHUNK,), jnp.int32),
        pltpu.VMEM_SHARED((NPAD, D), jnp.float32),
    ] + [pltpu.SemaphoreType.DMA] * 5,
)(_sc_edge_kernel)


def _elu(x):
    return jnp.where(x > 0, x, jnp.exp(jnp.minimum(x, 0.0)) - 1.0)


BN = 512
GRID = NPAD // BN


def _tc_mlp_kernel(xpar, xch, xop, xself, w1, w2, w3, wc1, wc2, wc3, ball,
                   out_ref):
    i = pl.program_id(0)
    f32 = jnp.float32

    def mlp(x, j):
        h = _elu(jnp.dot(x, w1[j], preferred_element_type=f32) + ball[j, 0])
        h = _elu(jnp.dot(h, w2[j], preferred_element_type=f32) + ball[j, 1])
        return jnp.dot(h, w3[j], preferred_element_type=f32) + ball[j, 2]

    e_par = mlp(xpar[...], 0)
    e_ch = mlp(xch[...], 1)
    e_op = mlp(xop[...], 2)
    e_self = mlp(xself[...], 3)

    wc1v = wc1[...]
    h = (jnp.dot(e_par, wc1v[0:128], preferred_element_type=f32)
         + jnp.dot(e_ch, wc1v[128:256], preferred_element_type=f32)
         + jnp.dot(e_op, wc1v[256:384], preferred_element_type=f32)
         + jnp.dot(e_self, wc1v[384:512], preferred_element_type=f32)
         + ball[4, 0])
    h = _elu(h)
    h = _elu(jnp.dot(h, wc2[...], preferred_element_type=f32) + ball[4, 1])
    y = jnp.dot(h, wc3[...], preferred_element_type=f32) + ball[4, 2]

    row = i * BN + lax.broadcasted_iota(jnp.int32, (BN, D), 0)
    out_ref[...] = jnp.where(row == (N - 1), 0.0, y)


def _tc_call(xpar, xch, xop, xself, w1, w2, w3, wc1, wc2, wc3, ball):
    full3 = pl.BlockSpec((4, D, D), lambda i: (0, 0, 0))
    blk = pl.BlockSpec((BN, D), lambda i: (i, 0))
    return pl.pallas_call(
        _tc_mlp_kernel,
        grid=(GRID,),
        in_specs=[blk, blk, blk, blk,
                  full3, full3, full3,
                  pl.BlockSpec((4 * D, D), lambda i: (0, 0)),
                  pl.BlockSpec((D, D), lambda i: (0, 0)),
                  pl.BlockSpec((D, D), lambda i: (0, 0)),
                  pl.BlockSpec((5, 3, D), lambda i: (0, 0, 0))],
        out_specs=blk,
        out_shape=jax.ShapeDtypeStruct((NPAD, D), jnp.float32),
    )(xpar, xch, xop, xself, w1, w2, w3, wc1, wc2, wc3, ball)


def kernel(items, operations, parents, item_assembly_edge_index,
           operation_assembly_edge_index, self_p, parent_p, children_p,
           ops_p, comb_p):
    i32 = jnp.int32
    table = jnp.concatenate([items, operations], axis=0)

    ii = item_assembly_edge_index.astype(i32)
    oi = operation_assembly_edge_index.astype(i32)
    pad_e = E_PAD - E
    src_all = jnp.concatenate([
        ii[1], jnp.zeros((pad_e,), i32),
        oi[1] + N, jnp.zeros((pad_e,), i32),
    ]).reshape(NC * NS, CH_PER_TILE, CHUNK)
    dst_all = jnp.concatenate([
        ii[0], jnp.full((pad_e,), DUMP_ROW, i32),
        oi[0], jnp.full((pad_e,), DUMP_ROW, i32),
    ]).reshape(NC * NS, CH_PER_TILE, CHUNK)
    eidx_hbm = jnp.stack([src_all, dst_all], axis=2).reshape(
        NC * NS, NIB, IB, 2, CHUNK)
    par_idx = jnp.concatenate([parents.astype(i32),
                               jnp.zeros((PPAD - N,), i32)])
    zinit = jnp.zeros((NPAD, D), jnp.float32)

    aggc, aggo, par_rows = _sc_call(table, eidx_hbm, par_idx, zinit)
    par_rows = par_rows[:NPAD]

    w1 = jnp.stack([parent_p[0], children_p[0], ops_p[0], self_p[0]])
    w2 = jnp.stack([parent_p[2], children_p[2], ops_p[2], self_p[2]])
    w3 = jnp.stack([parent_p[4], children_p[4], ops_p[4], self_p[4]])
    ball = jnp.stack([
        jnp.stack([parent_p[1], parent_p[3], parent_p[5]]),
        jnp.stack([children_p[1], children_p[3], children_p[5]]),
        jnp.stack([ops_p[1], ops_p[3], ops_p[5]]),
        jnp.stack([self_p[1], self_p[3], self_p[5]]),
        jnp.stack([comb_p[1], comb_p[3], comb_p[5]]),
    ])

    items_pad = jnp.pad(items, ((0, NPAD - N), (0, 0)))
    y = _tc_call(par_rows, aggc, aggo, items_pad,
                 w1, w2, w3, comb_p[0], comb_p[2], comb_p[4], ball)
    return y[:N]
